# baseline (speedup 1.0000x reference)
import functools

import jax
import jax.numpy as jnp
from jax import lax
from jax.experimental import pallas as pl
from jax.experimental.pallas import tpu as pltpu
from jax.experimental.pallas import tpu_sc as plsc

_NC = 2
_NS = 16
_K = 128
_KB = 1
_LAYERS = 3


def _row_block(n):
    for r in range(1024, 7, -8):
        if n % r == 0:
            return r
    return None


def _mm_split_body(x_ref, w_ref, o0_ref, o1_ref):
    res = jnp.dot(x_ref[...], w_ref[...], preferred_element_type=jnp.float32)
    fh = res.shape[1] // 2
    o0_ref[...] = res[:, :fh]
    o1_ref[...] = res[:, fh:]


def _matmul_split(x, w):
    n, f = x.shape
    f2 = w.shape[1]
    fh = f2 // 2
    r = _row_block(n)
    return pl.pallas_call(
        _mm_split_body,
        grid=(n // r,),
        in_specs=[pl.BlockSpec((r, f), lambda i: (i, 0)),
                  pl.BlockSpec((f, f2), lambda i: (0, 0))],
        out_specs=[pl.BlockSpec((r, fh), lambda i: (i, 0)),
                   pl.BlockSpec((r, fh), lambda i: (i, 0))],
        out_shape=[jax.ShapeDtypeStruct((n, fh), jnp.float32),
                   jax.ShapeDtypeStruct((n, fh), jnp.float32)],
    )(x, w)


def _gru_math(p_ref, h_ref, wih_ref, whh_ref, bih_ref, bhh_ref):
    agg = jnp.concatenate([p_ref[0], p_ref[1]], axis=-1)
    h = h_ref[...]
    f = h.shape[1]
    gi = jnp.dot(agg, wih_ref[...], preferred_element_type=jnp.float32)
    gi = gi + bih_ref[...]
    gh = jnp.dot(h, whh_ref[...], preferred_element_type=jnp.float32)
    gh = gh + bhh_ref[...]
    r = jax.nn.sigmoid(gi[:, :f] + gh[:, :f])
    z = jax.nn.sigmoid(gi[:, f:2 * f] + gh[:, f:2 * f])
    nn = jnp.tanh(gi[:, 2 * f:] + r * gh[:, 2 * f:])
    return (1.0 - z) * nn + z * h


def _gru_body(p_ref, h_ref, wih_ref, whh_ref, bih_ref, bhh_ref, ho_ref):
    ho_ref[...] = _gru_math(p_ref, h_ref, wih_ref, whh_ref, bih_ref, bhh_ref)


def _gru_m_body(p_ref, h_ref, wih_ref, whh_ref, bih_ref, bhh_ref, wfc_ref,
                ho_ref, mo0_ref, mo1_ref):
    hn = _gru_math(p_ref, h_ref, wih_ref, whh_ref, bih_ref, bhh_ref)
    ho_ref[...] = hn
    m = jnp.dot(hn, wfc_ref[...], preferred_element_type=jnp.float32)
    fh = m.shape[1] // 2
    mo0_ref[...] = m[:, :fh]
    mo1_ref[...] = m[:, fh:]


def _gru(parts, h, wih_t, whh_t, bih, bhh, wfc_t):
    n, f = h.shape
    fh = f // 2
    r = _row_block(n)
    g = n // r
    in_specs = [
        pl.BlockSpec((_NC, r, fh), lambda i: (0, i, 0)),
        pl.BlockSpec((r, f), lambda i: (i, 0)),
        pl.BlockSpec((f, 3 * f), lambda i: (0, 0)),
        pl.BlockSpec((f, 3 * f), lambda i: (0, 0)),
        pl.BlockSpec((1, 3 * f), lambda i: (0, 0)),
        pl.BlockSpec((1, 3 * f), lambda i: (0, 0)),
    ]
    args = [parts, h, wih_t, whh_t, bih, bhh]
    if wfc_t is None:
        return pl.pallas_call(
            _gru_body,
            grid=(g,),
            in_specs=in_specs,
            out_specs=pl.BlockSpec((r, f), lambda i: (i, 0)),
            out_shape=jax.ShapeDtypeStruct((n, f), jnp.float32),
        )(*args)
    in_specs.append(pl.BlockSpec((f, f), lambda i: (0, 0)))
    args.append(wfc_t)
    return pl.pallas_call(
        _gru_m_body,
        grid=(g,),
        in_specs=in_specs,
        out_specs=[pl.BlockSpec((r, f), lambda i: (i, 0)),
                   pl.BlockSpec((r, fh), lambda i: (i, 0)),
                   pl.BlockSpec((r, fh), lambda i: (i, 0))],
        out_shape=[jax.ShapeDtypeStruct((n, f), jnp.float32),
                   jax.ShapeDtypeStruct((n, fh), jnp.float32),
                   jax.ShapeDtypeStruct((n, fh), jnp.float32)],
    )(*args)


@functools.cache
def _sc_agg(n_acc, nch, fh):
    mesh = plsc.VectorSubcoreMesh(core_axis_name="c", subcore_axis_name="s",
                                  num_cores=_NC, num_subcores=_NS)
    rows_per_tile = n_acc // _NS
    nzch = rows_per_tile // _K

    @functools.partial(
        pl.kernel,
        mesh=mesh,
        compiler_params=pltpu.CompilerParams(use_tc_tiling_on_sc=False),
        out_type=jax.ShapeDtypeStruct((_NC, n_acc, fh), jnp.float32),
        scratch_types=[
            pltpu.VMEM((nch, _KB * _K), jnp.int32),
            pltpu.VMEM((nch, _KB * _K), jnp.int32),
            [pltpu.VMEM((_KB * _K, fh), jnp.float32)] * 3,
            pltpu.VMEM((_K, fh), jnp.float32),
            pltpu.VMEM_SHARED((n_acc, fh), jnp.float32),
            [pltpu.SemaphoreType.DMA] * 3,
            [pltpu.SemaphoreType.DMA] * 3,
            pltpu.SemaphoreType.DMA,
            pltpu.SemaphoreType.DMA,
        ],
    )
    def k(m0_hbm, m1_hbm, src_hbm, dst_hbm, z_hbm, out_hbm,
          src_v, dst_v, rows_v, stage_v, acc_sh,
          gsem, ssem, isem0, isem1):
        c = lax.axis_index("c")
        s = lax.axis_index("s")
        base = s * rows_per_tile

        pltpu.async_copy(src_hbm.at[s], src_v, isem0)
        pltpu.async_copy(dst_hbm.at[s], dst_v, isem1)
        pltpu.sync_copy(z_hbm, stage_v)
        pltpu.make_async_copy(src_hbm.at[s], src_v, isem0).wait()
        pltpu.make_async_copy(dst_hbm.at[s], dst_v, isem1).wait()

        def gstart(j, buf, sem):
            @pl.when(c == 0)
            def _():
                pltpu.async_copy(m0_hbm.at[src_v.at[j]], buf, sem)

            @pl.when(c == 1)
            def _():
                pltpu.async_copy(m1_hbm.at[src_v.at[j]], buf, sem)

        def gwait(buf, sem):
            pltpu.make_async_copy(m0_hbm.at[src_v.at[0]], buf, sem).wait()

        gstart(0, rows_v[0], gsem[0])
        gstart(1, rows_v[1], gsem[1])

        def zfire(i, carry):
            pltpu.async_copy(stage_v, acc_sh.at[pl.ds(base + i * _K, _K)],
                             isem0)
            return carry
        lax.fori_loop(0, nzch, zfire, 0)

        def zdrain(i, carry):
            pltpu.make_async_copy(stage_v, acc_sh.at[pl.ds(base, _K)],
                                  isem0).wait()
            return carry
        lax.fori_loop(0, nzch, zdrain, 0)
        plsc.subcore_barrier()

        def sstart(j, buf, sem):
            pltpu.async_copy(buf, acc_sh.at[dst_v.at[j]], sem, add=True)

        def swait(buf, sem):
            pltpu.make_async_copy(buf, acc_sh.at[dst_v.at[0]], sem).wait()

        def trip(t, carry):
            j0 = 3 * t
            for x in range(3):
                j = j0 + x
                gwait(rows_v[x], gsem[x])
                sstart(j, rows_v[x], ssem[x])
                nx = (x + 2) % 3

                if x == 0:
                    @pl.when((j + 2 < nch) & (t > 0))
                    def _():
                        swait(rows_v[nx], ssem[nx])
                        gstart(j + 2, rows_v[nx], gsem[nx])

                    @pl.when((j + 2 < nch) & (t == 0))
                    def _():
                        gstart(j + 2, rows_v[nx], gsem[nx])
                else:
                    @pl.when(j + 2 < nch)
                    def _(nx=nx, j=j):
                        swait(rows_v[nx], ssem[nx])
                        gstart(j + 2, rows_v[nx], gsem[nx])
            return carry
        lax.fori_loop(0, nch // 3, trip, 0)
        for x in range(3):
            swait(rows_v[x], ssem[x])
        plsc.subcore_barrier()

        def ofire(i, carry):
            pltpu.async_copy(acc_sh.at[pl.ds(base + i * _K, _K)],
                             out_hbm.at[c].at[pl.ds(base + i * _K, _K)],
                             isem0)
            return carry
        lax.fori_loop(0, nzch, ofire, 0)

        def odrain(i, carry):
            pltpu.make_async_copy(acc_sh.at[pl.ds(base, _K)],
                                  out_hbm.at[c].at[pl.ds(base, _K)],
                                  isem0).wait()
            return carry
        lax.fori_loop(0, nzch, odrain, 0)

    return k


def kernel(feat, edge_index, W_fc, W_ih, W_hh, b_ih, b_hh):
    n, f_in = feat.shape
    f = W_fc.shape[0]
    h = feat if f_in == f else jnp.pad(feat, ((0, 0), (0, f - f_in)))
    wfc_t = W_fc.T
    wih_t = W_ih.T
    whh_t = W_hh.T
    bih = b_ih.reshape(1, -1)
    bhh = b_hh.reshape(1, -1)

    src = edge_index[0].astype(jnp.int32)
    dst = edge_index[1].astype(jnp.int32)
    e = src.shape[0]
    blk = _KB * _K
    per_s = -(-e // (_NS * 6 * blk)) * 6 * blk
    nch = per_s // blk
    pad = _NS * per_s - e
    src_r = jnp.concatenate([src, jnp.zeros((pad,), jnp.int32)])
    dst_r = jnp.concatenate([dst, jnp.full((pad,), n, jnp.int32)])
    src_r = src_r.reshape(_NS, nch, _KB * _K)
    dst_r = dst_r.reshape(_NS, nch, _KB * _K)
    n_acc = -(-(n + 1) // (_NS * _K)) * (_NS * _K)
    zeros = jnp.zeros((_K, f // 2), jnp.float32)

    sc = _sc_agg(n_acc, nch, f // 2)

    m0, m1 = _matmul_split(h, wfc_t)
    for layer in range(_LAYERS):
        parts = sc(m0, m1, src_r, dst_r, zeros)
        if layer == _LAYERS - 1:
            h = _gru(parts, h, wih_t, whh_t, bih, bhh, None)
        else:
            h, m0, m1 = _gru(parts, h, wih_t, whh_t, bih, bhh, wfc_t)
    return h


# speedup vs baseline: 1.8939x; 1.8939x over previous
import functools

import jax
import jax.numpy as jnp
from jax import lax
from jax.experimental import pallas as pl
from jax.experimental.pallas import tpu as pltpu
from jax.experimental.pallas import tpu_sc as plsc

_NC = 2
_NS = 16
_K = 128
_KB = 1
_LAYERS = 3


def _row_block(n):
    for r in range(1024, 7, -8):
        if n % r == 0:
            return r
    return None


def _mm_split_body(x_ref, w_ref, o0_ref, o1_ref):
    res = jnp.dot(x_ref[...], w_ref[...], preferred_element_type=jnp.float32)
    fh = res.shape[1] // 2
    o0_ref[...] = res[:, :fh]
    o1_ref[...] = res[:, fh:]


def _matmul_split(x, w):
    n, f = x.shape
    f2 = w.shape[1]
    fh = f2 // 2
    r = _row_block(n)
    return pl.pallas_call(
        _mm_split_body,
        grid=(n // r,),
        in_specs=[pl.BlockSpec((r, f), lambda i: (i, 0)),
                  pl.BlockSpec((f, f2), lambda i: (0, 0))],
        out_specs=[pl.BlockSpec((r, fh), lambda i: (i, 0)),
                   pl.BlockSpec((r, fh), lambda i: (i, 0))],
        out_shape=[jax.ShapeDtypeStruct((n, fh), jnp.float32),
                   jax.ShapeDtypeStruct((n, fh), jnp.float32)],
    )(x, w)


def _gru_math(p_ref, h_ref, wih_ref, whh_ref, bih_ref, bhh_ref):
    agg = jnp.concatenate([p_ref[0], p_ref[1]], axis=-1)
    h = h_ref[...]
    f = h.shape[1]
    gi = jnp.dot(agg, wih_ref[...], preferred_element_type=jnp.float32)
    gi = gi + bih_ref[...]
    gh = jnp.dot(h, whh_ref[...], preferred_element_type=jnp.float32)
    gh = gh + bhh_ref[...]
    r = jax.nn.sigmoid(gi[:, :f] + gh[:, :f])
    z = jax.nn.sigmoid(gi[:, f:2 * f] + gh[:, f:2 * f])
    nn = jnp.tanh(gi[:, 2 * f:] + r * gh[:, 2 * f:])
    return (1.0 - z) * nn + z * h


def _gru_body(p_ref, h_ref, wih_ref, whh_ref, bih_ref, bhh_ref, ho_ref):
    ho_ref[...] = _gru_math(p_ref, h_ref, wih_ref, whh_ref, bih_ref, bhh_ref)


def _gru_m_body(p_ref, h_ref, wih_ref, whh_ref, bih_ref, bhh_ref, wfc_ref,
                ho_ref, mo0_ref, mo1_ref):
    hn = _gru_math(p_ref, h_ref, wih_ref, whh_ref, bih_ref, bhh_ref)
    ho_ref[...] = hn
    m = jnp.dot(hn, wfc_ref[...], preferred_element_type=jnp.float32)
    fh = m.shape[1] // 2
    mo0_ref[...] = m[:, :fh]
    mo1_ref[...] = m[:, fh:]


def _gru(parts, h, wih_t, whh_t, bih, bhh, wfc_t):
    n, f = h.shape
    fh = f // 2
    r = _row_block(n)
    g = n // r
    in_specs = [
        pl.BlockSpec((_NC, r, fh), lambda i: (0, i, 0)),
        pl.BlockSpec((r, f), lambda i: (i, 0)),
        pl.BlockSpec((f, 3 * f), lambda i: (0, 0)),
        pl.BlockSpec((f, 3 * f), lambda i: (0, 0)),
        pl.BlockSpec((1, 3 * f), lambda i: (0, 0)),
        pl.BlockSpec((1, 3 * f), lambda i: (0, 0)),
    ]
    args = [parts, h, wih_t, whh_t, bih, bhh]
    if wfc_t is None:
        return pl.pallas_call(
            _gru_body,
            grid=(g,),
            in_specs=in_specs,
            out_specs=pl.BlockSpec((r, f), lambda i: (i, 0)),
            out_shape=jax.ShapeDtypeStruct((n, f), jnp.float32),
        )(*args)
    in_specs.append(pl.BlockSpec((f, f), lambda i: (0, 0)))
    args.append(wfc_t)
    return pl.pallas_call(
        _gru_m_body,
        grid=(g,),
        in_specs=in_specs,
        out_specs=[pl.BlockSpec((r, f), lambda i: (i, 0)),
                   pl.BlockSpec((r, fh), lambda i: (i, 0)),
                   pl.BlockSpec((r, fh), lambda i: (i, 0))],
        out_shape=[jax.ShapeDtypeStruct((n, f), jnp.float32),
                   jax.ShapeDtypeStruct((n, fh), jnp.float32),
                   jax.ShapeDtypeStruct((n, fh), jnp.float32)],
    )(*args)


@functools.cache
def _sc_agg(n_acc, nch, fh):
    mesh = plsc.VectorSubcoreMesh(core_axis_name="c", subcore_axis_name="s",
                                  num_cores=_NC, num_subcores=_NS)
    rows_per_tile = n_acc // _NS
    nzch = rows_per_tile // _K

    @functools.partial(
        pl.kernel,
        mesh=mesh,
        compiler_params=pltpu.CompilerParams(use_tc_tiling_on_sc=False),
        out_type=jax.ShapeDtypeStruct((_NC, n_acc, fh), jnp.float32),
        scratch_types=[
            pltpu.VMEM((nch, _KB * _K), jnp.int32),
            pltpu.VMEM((nch, _KB * _K), jnp.int32),
            pltpu.VMEM((_KB * _K, fh), jnp.float32),
            pltpu.VMEM((_KB * _K, fh), jnp.float32),
            pltpu.VMEM((_K, fh), jnp.float32),
            pltpu.VMEM_SHARED((n_acc, fh), jnp.float32),
            pltpu.SemaphoreType.DMA,
            pltpu.SemaphoreType.DMA,
            pltpu.SemaphoreType.DMA,
            pltpu.SemaphoreType.DMA,
        ],
    )
    def k(m0_hbm, m1_hbm, src_hbm, dst_hbm, z_hbm, out_hbm,
          src_v, dst_v, rows0_v, rows1_v, stage_v, acc_sh,
          sem0, sem1, isem0, isem1):
        c = lax.axis_index("c")
        s = lax.axis_index("s")
        base = s * rows_per_tile

        pltpu.async_copy(src_hbm.at[s], src_v, isem0)
        pltpu.async_copy(dst_hbm.at[s], dst_v, isem1)
        pltpu.sync_copy(z_hbm, stage_v)
        pltpu.make_async_copy(src_hbm.at[s], src_v, isem0).wait()
        pltpu.make_async_copy(dst_hbm.at[s], dst_v, isem1).wait()

        def gstart(j, buf, sem):
            @pl.when(c == 0)
            def _():
                pltpu.async_copy(m0_hbm.at[src_v.at[j]], buf, sem)

            @pl.when(c == 1)
            def _():
                pltpu.async_copy(m1_hbm.at[src_v.at[j]], buf, sem)

        def gwait(buf, sem):
            pltpu.make_async_copy(m0_hbm.at[src_v.at[0]], buf, sem).wait()

        gstart(0, rows0_v, sem0)

        def zfire(i, carry):
            pltpu.async_copy(stage_v, acc_sh.at[pl.ds(base + i * _K, _K)],
                             isem0)
            return carry
        lax.fori_loop(0, nzch, zfire, 0)

        def zdrain(i, carry):
            pltpu.make_async_copy(stage_v, acc_sh.at[pl.ds(base, _K)],
                                  isem0).wait()
            return carry
        lax.fori_loop(0, nzch, zdrain, 0)
        plsc.subcore_barrier()


        def pair(p, carry):
            a = 2 * p
            gstart(a + 1, rows1_v, sem1)
            gwait(rows0_v, sem0)
            pltpu.sync_copy(rows0_v, acc_sh.at[dst_v.at[a]], add=True)

            @pl.when(a + 2 < nch)
            def _():
                gstart(a + 2, rows0_v, sem0)

            gwait(rows1_v, sem1)
            pltpu.sync_copy(rows1_v, acc_sh.at[dst_v.at[a + 1]], add=True)
            return carry
        lax.fori_loop(0, nch // 2, pair, 0)
        plsc.subcore_barrier()

        def ofire(i, carry):
            pltpu.async_copy(acc_sh.at[pl.ds(base + i * _K, _K)],
                             out_hbm.at[c].at[pl.ds(base + i * _K, _K)],
                             sem0)
            return carry
        lax.fori_loop(0, nzch, ofire, 0)

        def odrain(i, carry):
            pltpu.make_async_copy(acc_sh.at[pl.ds(base, _K)],
                                  out_hbm.at[c].at[pl.ds(base, _K)],
                                  sem0).wait()
            return carry
        lax.fori_loop(0, nzch, odrain, 0)

    return k


def kernel(feat, edge_index, W_fc, W_ih, W_hh, b_ih, b_hh):
    n, f_in = feat.shape
    f = W_fc.shape[0]
    h = feat if f_in == f else jnp.pad(feat, ((0, 0), (0, f - f_in)))
    wfc_t = W_fc.T
    wih_t = W_ih.T
    whh_t = W_hh.T
    bih = b_ih.reshape(1, -1)
    bhh = b_hh.reshape(1, -1)

    src = edge_index[0].astype(jnp.int32)
    dst = edge_index[1].astype(jnp.int32)
    e = src.shape[0]
    blk = _KB * _K
    per_s = -(-e // (_NS * 2 * blk)) * 2 * blk
    nch = per_s // blk
    pad = _NS * per_s - e
    src_r = jnp.concatenate([src, jnp.zeros((pad,), jnp.int32)])
    dst_r = jnp.concatenate([dst, jnp.full((pad,), n, jnp.int32)])
    src_r = src_r.reshape(_NS, nch, _KB * _K)
    dst_r = dst_r.reshape(_NS, nch, _KB * _K)
    n_acc = -(-(n + 1) // (_NS * _K)) * (_NS * _K)
    zeros = jnp.zeros((_K, f // 2), jnp.float32)

    sc = _sc_agg(n_acc, nch, f // 2)

    m0, m1 = _matmul_split(h, wfc_t)
    for layer in range(_LAYERS):
        parts = sc(m0, m1, src_r, dst_r, zeros)
        if layer == _LAYERS - 1:
            h = _gru(parts, h, wih_t, whh_t, bih, bhh, None)
        else:
            h, m0, m1 = _gru(parts, h, wih_t, whh_t, bih, bhh, wfc_t)
    return h
